# Initial kernel scaffold; baseline (speedup 1.0000x reference)
import jax
import jax.numpy as jnp
from jax.experimental import pallas as pl


def kernel(x, edge_index, batch, c1_w1, c1_b1, c1_g, c1_be, c1_w2, c1_b2, l2_w1, l2_b1, l2_g, l2_be, l2_w2, l2_b2, l3_w1, l3_b1, l3_g, l3_be, l3_w2, l3_b2, cls_w1, cls_b1, cls_w2, cls_b2, cls_w3, cls_b3):
    raise NotImplementedError("write your pallas kernel here")


# speedup vs baseline: 5.5516x; 5.5516x over previous
import functools

import jax
import jax.numpy as jnp
from jax import lax
from jax.experimental import pallas as pl
from jax.experimental.pallas import tpu as pltpu
from jax.experimental.pallas import tpu_sc as plsc

N = 10000
NP = 10240
E = 320000
D = 128
G = 64
C = 16

NC = 2
NS = 16
NW = NC * NS
EPW = E // NW
CHUNK = 128
NCHUNK = EPW // CHUNK
TAIL = EPW - NCHUNK * CHUNK
RPT = NP // NS

RT = 1024
T = NP // RT


def _sc_segsum_body(h_hbm, src_hbm, dst_hbm, zero_hbm, out_hbm,
                    sidx, didx, rows, sidx_t, didx_t, rows_t, acc, sem):
    c = lax.axis_index("c")
    s = lax.axis_index("s")
    wid = s * NC + c

    pltpu.sync_copy(zero_hbm.at[pl.ds(s * RPT, RPT)], acc.at[pl.ds(s * RPT, RPT)])
    plsc.subcore_barrier()

    base0 = wid * EPW

    def chunk(i, carry):
        base = base0 + i * CHUNK
        pltpu.sync_copy(src_hbm.at[pl.ds(base, CHUNK)], sidx)
        pltpu.sync_copy(dst_hbm.at[pl.ds(base, CHUNK)], didx)
        pltpu.async_copy(h_hbm.at[sidx], rows, sem).wait()
        pltpu.sync_copy(rows, acc.at[didx], add=True)
        return carry

    lax.fori_loop(0, NCHUNK, chunk, 0)

    tbase = base0 + NCHUNK * CHUNK
    pltpu.sync_copy(src_hbm.at[pl.ds(tbase, TAIL)], sidx_t)
    pltpu.sync_copy(dst_hbm.at[pl.ds(tbase, TAIL)], didx_t)
    pltpu.async_copy(h_hbm.at[sidx_t], rows_t, sem).wait()
    pltpu.sync_copy(rows_t, acc.at[didx_t], add=True)

    plsc.subcore_barrier()
    pltpu.sync_copy(acc.at[pl.ds(s * RPT, RPT)], out_hbm.at[c, pl.ds(s * RPT, RPT)])


_sc_segsum = pl.kernel(
    _sc_segsum_body,
    out_type=jax.ShapeDtypeStruct((NC, NP, D), jnp.float32),
    mesh=plsc.VectorSubcoreMesh(core_axis_name="c", subcore_axis_name="s"),
    scratch_types=[
        pltpu.VMEM((CHUNK,), jnp.int32),
        pltpu.VMEM((CHUNK,), jnp.int32),
        pltpu.VMEM((CHUNK, D), jnp.float32),
        pltpu.VMEM((TAIL,), jnp.int32),
        pltpu.VMEM((TAIL,), jnp.int32),
        pltpu.VMEM((TAIL, D), jnp.float32),
        pltpu.VMEM_SHARED((NP, D), jnp.float32),
        pltpu.SemaphoreType.DMA,
    ],
)


def _layer_body(h_ref, a0_ref, a1_ref, w1_ref, w2_ref, b2_ref, g_ref, be_ref,
                batch_ref, hout_ref, p_ref, y1_scr, s1_scr, s2_scr):
    ph = pl.program_id(0)
    t = pl.program_id(1)

    @pl.when(jnp.logical_and(ph == 0, t == 0))
    def _init():
        s1_scr[...] = jnp.zeros_like(s1_scr)
        s2_scr[...] = jnp.zeros_like(s2_scr)

    @pl.when(ph == 0)
    def _phase0():
        z = h_ref[...] + a0_ref[...] + a1_ref[...]
        y1 = jnp.dot(z, w1_ref[...], preferred_element_type=jnp.float32)
        off = pl.multiple_of(t * RT, RT)
        y1_scr[pl.ds(off, RT), :] = y1
        s1_scr[...] += jnp.sum(y1, axis=0, keepdims=True)
        s2_scr[...] += jnp.sum(y1 * y1, axis=0, keepdims=True)

    @pl.when(ph == 1)
    def _phase1():
        nreal = jnp.float32(N)
        mean = s1_scr[...] / nreal
        var = s2_scr[...] / nreal - mean * mean
        inv = lax.rsqrt(var + 1e-5)
        off = pl.multiple_of(t * RT, RT)
        y1 = y1_scr[pl.ds(off, RT), :]
        yn = (y1 - mean) * (inv * g_ref[...]) + be_ref[...]
        yn = jnp.maximum(yn, 0.0)
        y2 = jnp.dot(yn, w2_ref[...], preferred_element_type=jnp.float32)
        y2 = jnp.maximum(y2 + b2_ref[...], 0.0)
        row = lax.broadcasted_iota(jnp.int32, (RT, D), 0) + t * RT
        y2 = jnp.where(row < N, y2, 0.0)
        hout_ref[...] = y2
        bt = batch_ref[0, 0, :]
        gi = lax.broadcasted_iota(jnp.int32, (G, RT), 0)
        oh = (gi == bt[None, :]).astype(jnp.float32)
        ptile = jnp.dot(oh, y2, preferred_element_type=jnp.float32)

        @pl.when(t == 0)
        def _():
            p_ref[...] = ptile

        @pl.when(t > 0)
        def _():
            p_ref[...] += ptile


def _tc_layer(h, a0, a1, w1, w2, b2, g, be, batch3d):
    return pl.pallas_call(
        _layer_body,
        grid=(2, T),
        in_specs=[
            pl.BlockSpec((RT, D), lambda p, t: ((1 - p) * t, 0)),
            pl.BlockSpec((RT, D), lambda p, t: ((1 - p) * t, 0)),
            pl.BlockSpec((RT, D), lambda p, t: ((1 - p) * t, 0)),
            pl.BlockSpec((D, D), lambda p, t: (0, 0)),
            pl.BlockSpec((D, D), lambda p, t: (0, 0)),
            pl.BlockSpec((1, D), lambda p, t: (0, 0)),
            pl.BlockSpec((1, D), lambda p, t: (0, 0)),
            pl.BlockSpec((1, D), lambda p, t: (0, 0)),
            pl.BlockSpec((1, 1, RT), lambda p, t: (t, 0, 0)),
        ],
        out_specs=[
            pl.BlockSpec((RT, D), lambda p, t: (p * t, 0)),
            pl.BlockSpec((G, D), lambda p, t: (0, 0)),
        ],
        out_shape=[
            jax.ShapeDtypeStruct((NP, D), jnp.float32),
            jax.ShapeDtypeStruct((G, D), jnp.float32),
        ],
        scratch_shapes=[
            pltpu.VMEM((NP, D), jnp.float32),
            pltpu.VMEM((1, D), jnp.float32),
            pltpu.VMEM((1, D), jnp.float32),
        ],
        compiler_params=pltpu.CompilerParams(
            dimension_semantics=("arbitrary", "arbitrary")),
    )(h, a0, a1, w1, w2, b2, g, be, batch3d)


def _cls_body(p1_ref, p2_ref, p3_ref, w1a_ref, w1b_ref, w1c_ref, b1_ref,
              w2_ref, b2_ref, w3_ref, b3_ref, out_ref):
    z1 = (jnp.dot(p1_ref[...], w1a_ref[...], preferred_element_type=jnp.float32)
          + jnp.dot(p2_ref[...], w1b_ref[...], preferred_element_type=jnp.float32)
          + jnp.dot(p3_ref[...], w1c_ref[...], preferred_element_type=jnp.float32)
          + b1_ref[...])
    z1 = jnp.maximum(z1, 0.0)
    z2 = jnp.maximum(jnp.dot(z1, w2_ref[...], preferred_element_type=jnp.float32)
                     + b2_ref[...], 0.0)
    out_ref[...] = (jnp.dot(z2, w3_ref[...], preferred_element_type=jnp.float32)
                    + b3_ref[...])


def _tc_cls(p1, p2, p3, w1a, w1b, w1c, b1, w2p, b2p, w3p, b3p):
    return pl.pallas_call(
        _cls_body,
        out_shape=jax.ShapeDtypeStruct((G, D), jnp.float32),
    )(p1, p2, p3, w1a, w1b, w1c, b1, w2p, b2p, w3p, b3p)


def kernel(x, edge_index, batch,
           c1_w1, c1_b1, c1_g, c1_be, c1_w2, c1_b2,
           l2_w1, l2_b1, l2_g, l2_be, l2_w2, l2_b2,
           l3_w1, l3_b1, l3_g, l3_be, l3_w2, l3_b2,
           cls_w1, cls_b1, cls_w2, cls_b2, cls_w3, cls_b3):
    src = edge_index[0]
    dst = edge_index[1]
    zero = jnp.zeros((NP, D), jnp.float32)
    xp = jnp.pad(x, ((0, NP - N), (0, 0)))
    batch3d = jnp.pad(batch, (0, NP - N), constant_values=G).reshape(T, 1, RT)

    pools = []
    h = xp
    for (w1, w2, b2, g, be) in (
        (c1_w1, c1_w2, c1_b2, c1_g, c1_be),
        (l2_w1, l2_w2, l2_b2, l2_g, l2_be),
        (l3_w1, l3_w2, l3_b2, l3_g, l3_be),
    ):
        agg = _sc_segsum(h, src, dst, zero)
        h, p = _tc_layer(h, agg[0], agg[1], w1, w2,
                         b2.reshape(1, D), g.reshape(1, D), be.reshape(1, D),
                         batch3d)
        pools.append(p)

    w1a, w1b, w1c = cls_w1[:D], cls_w1[D:2 * D], cls_w1[2 * D:]
    b1 = cls_b1.reshape(1, D)
    w2p = jnp.pad(cls_w2, ((0, 0), (0, D - cls_w2.shape[1])))
    b2p = jnp.pad(cls_b2, (0, D - cls_b2.shape[0])).reshape(1, D)
    w3p = jnp.pad(cls_w3, ((0, D - cls_w3.shape[0]), (0, D - cls_w3.shape[1])))
    b3p = jnp.pad(cls_b3, (0, D - cls_b3.shape[0])).reshape(1, D)

    outp = _tc_cls(pools[0], pools[1], pools[2],
                   w1a, w1b, w1c, b1, w2p, b2p, w3p, b3p)
    return outp[:, :C]
